# baseline (speedup 1.0000x reference)
import functools

import jax
import jax.numpy as jnp
from jax import lax
from jax.experimental import pallas as pl
from jax.experimental.pallas import tpu as pltpu
from jax.experimental.pallas import tpu_sc as plsc

_B, _S, _H = 4, 8192, 32
_NW = 32
_CHUNK = _S // _NW


def _copy_body(x_hbm, o_hbm, buf, sem):
    w = lax.axis_index("s") * 2 + lax.axis_index("c")
    sl = pl.ds(w * _CHUNK, _CHUNK)
    pltpu.make_async_copy(x_hbm.at[:, sl], buf, sem).start()
    pltpu.make_async_copy(x_hbm.at[:, sl], buf, sem).wait()
    pltpu.make_async_copy(buf, o_hbm.at[:, sl], sem).start()
    pltpu.make_async_copy(buf, o_hbm.at[:, sl], sem).wait()


def kernel(inputs_embeds):
    mesh = plsc.VectorSubcoreMesh(core_axis_name="c", subcore_axis_name="s")
    k = pl.kernel(
        _copy_body,
        out_type=jax.ShapeDtypeStruct((_B, _S, _H), jnp.float32),
        mesh=mesh,
        scratch_types=[
            pltpu.VMEM((_B, _CHUNK, _H), jnp.float32),
            pltpu.SemaphoreType.DMA,
        ],
    )
    return k(inputs_embeds)
